# Initial kernel scaffold; baseline (speedup 1.0000x reference)
import jax
import jax.numpy as jnp
from jax.experimental import pallas as pl


def kernel(output, target):
    raise NotImplementedError("write your pallas kernel here")


# speedup vs baseline: 7.0334x; 7.0334x over previous
import functools

import jax
import jax.numpy as jnp
from jax import lax
from jax.experimental import pallas as pl
from jax.experimental.pallas import tpu as pltpu

_MXL = 100.0
_MNL = -100.0
_NP = 8
_NN = 64


def _loss_body(out_ref, tgt_ref, acc_ref):
    x = out_ref[...]
    r, l = x.shape
    tmask = tgt_ref[...] == 1.0
    sim_m = jnp.where(tmask, _MNL, x)
    sim_p = jnp.where(tmask, x, _MXL)

    iota = lax.broadcasted_iota(jnp.int32, (r, l), 1)
    sp_vals = []
    sp_cur = sim_p
    for _ in range(_NP):
        v = jnp.min(sp_cur, axis=1, keepdims=True)
        idx = jnp.min(jnp.where(sp_cur == v, iota, l), axis=1, keepdims=True)
        sp_vals.append(v)
        sp_cur = jnp.where(iota == idx, _MXL, sp_cur)

    u = lax.bitcast_convert_type(sim_m, jnp.uint32)
    topbit = jnp.uint32(0x80000000)
    k = jnp.where(u >= topbit, ~u, u | topbit)

    lo0 = jnp.zeros((r, 1), jnp.uint32)
    hi0 = jnp.full((r, 1), 0xFFFFFFFF, jnp.uint32)

    def bs_body(_, lohi):
        lo, hi = lohi
        d = hi - lo
        mid = lo + (d >> jnp.uint32(1)) + (d & jnp.uint32(1))
        cnt = jnp.sum((k >= mid).astype(jnp.int32), axis=1, keepdims=True)
        ge = cnt >= _NN
        return jnp.where(ge, mid, lo), jnp.where(ge, hi, mid - jnp.uint32(1))

    tkey, _ = lax.fori_loop(0, 32, bs_body, (lo0, hi0))

    setmask = k > tkey
    c = jnp.sum(setmask.astype(jnp.float32), axis=1, keepdims=True)
    u2 = jnp.where(tkey >= topbit, tkey & jnp.uint32(0x7FFFFFFF), ~tkey)
    tval = lax.bitcast_convert_type(u2, jnp.float32)
    maxrow = jnp.max(sim_m, axis=1, keepdims=True)

    y = sim_m + 1.0
    e = jnp.exp(y)
    setf = setmask.astype(jnp.float32)
    emask = jnp.where(setmask, e, 0.0)
    wmask = emask * y

    total = jnp.zeros((r, 1), jnp.float32)
    for j in range(_NP):
        sp = sp_vals[j]
        mp = jnp.maximum(maxrow + 1.0 - sp, 0.0)
        kp = jnp.exp(-sp - mp)
        pm = y > sp
        s2 = jnp.sum(jnp.where(pm, emask, 0.0), axis=1, keepdims=True)
        s1 = jnp.sum(jnp.where(pm, wmask, 0.0), axis=1, keepdims=True)
        cntp = jnp.sum(jnp.where(pm, setf, 0.0), axis=1, keepdims=True)
        num = kp * (s1 - sp * s2)
        den = kp * s2
        losst = jnp.maximum(tval + 1.0 - sp, 0.0)
        et = jnp.exp(losst - mp)
        cf = _NN - c
        tpos = losst > 0.0
        num = num + jnp.where(tpos, cf * et * losst, 0.0)
        den = den + jnp.where(tpos, cf * et, 0.0)
        zeros = (c - cntp) + jnp.where(tpos, 0.0, cf)
        den = den + zeros * jnp.exp(-100.0 - mp)
        anyl = tpos | (cntp > 0.0)
        den_safe = jnp.where(anyl, den, 1.0)
        total = total + jnp.where(anyl, num / den_safe, 0.0)

    @pl.when(pl.program_id(0) == 0)
    def _():
        acc_ref[...] = jnp.zeros_like(acc_ref)

    acc_ref[...] += jnp.sum(total, axis=(0, 1), keepdims=True)


@jax.jit
def kernel(output, target):
    b, l = output.shape
    r = 16 if b % 16 == 0 else 1
    grid = b // r
    acc = pl.pallas_call(
        _loss_body,
        grid=(grid,),
        in_specs=[
            pl.BlockSpec((r, l), lambda i: (i, 0)),
            pl.BlockSpec((r, l), lambda i: (i, 0)),
        ],
        out_specs=pl.BlockSpec((1, 1), lambda i: (0, 0)),
        out_shape=jax.ShapeDtypeStruct((1, 1), jnp.float32),
    )(output, target)
    return acc[0, 0] / (b * _NP * _NN)


# speedup vs baseline: 11.4457x; 1.6273x over previous
import functools

import jax
import jax.numpy as jnp
from jax import lax
from jax.experimental import pallas as pl
from jax.experimental.pallas import tpu as pltpu
from jax.experimental.pallas import tpu_sc as plsc

_MXL = 100.0
_MNL = -100.0
_NP = 8
_NN = 64
_BIG = 3.4e38
_L16 = 16


def _vec16(x):
    return jnp.full((_L16,), x, jnp.float32)


_IOTA = lambda: lax.broadcasted_iota(jnp.int32, (_L16,), 0)


def _extract(v, j, pad):
    return jnp.min(jnp.where(_IOTA() == j, v, pad))


def _sc_body(x_hbm, t_hbm, out_hbm, xbuf, tbuf, smbuf, spbuf, nbuf, pbuf, obuf):
    b = x_hbm.shape[0]
    l = x_hbm.shape[1]
    nvec = l // _L16
    ngrp = 4
    gvec = nvec // ngrp
    wid = lax.axis_index("s") * 2 + lax.axis_index("c")
    rpw = b // 32
    iota = _IOTA()

    def row_step(r, acc):
        row = wid * rpw + r
        pltpu.sync_copy(x_hbm.at[row], xbuf)
        pltpu.sync_copy(t_hbm.at[row], tbuf)

        gmaxs = []
        pmin = _vec16(_BIG)
        for g in range(ngrp):
            def p1(i, carry):
                gmax, pmin = carry
                xv = xbuf[pl.ds((g * gvec + i) * _L16, _L16)]
                tv = tbuf[pl.ds((g * gvec + i) * _L16, _L16)]
                m1 = tv == 1.0
                sm = jnp.where(m1, _MNL, xv)
                sp = jnp.where(m1, xv, _MXL)
                smbuf[pl.ds((g * gvec + i) * _L16, _L16)] = sm
                spbuf[pl.ds((g * gvec + i) * _L16, _L16)] = sp
                return jnp.maximum(gmax, sm), jnp.minimum(pmin, sp)

            gmax, pmin = lax.fori_loop(0, gvec, p1, (_vec16(-_BIG), pmin))
            gmaxs.append(gmax)
        tau_n = jnp.minimum(jnp.minimum(jnp.min(gmaxs[0]), jnp.min(gmaxs[1])),
                            jnp.minimum(jnp.min(gmaxs[2]), jnp.min(gmaxs[3])))
        tau_p = jnp.max(pmin)

        def p2(i, carry):
            offn, offp = carry
            sm = smbuf[pl.ds(i * _L16, _L16)]
            sp = spbuf[pl.ds(i * _L16, _L16)]
            mn = sm >= tau_n
            csn = plsc.cumsum(jnp.where(mn, 1, 0))
            plsc.store_scatter(nbuf, [offn + csn - 1], sm, mask=mn)
            mp = sp <= tau_p
            csp = plsc.cumsum(jnp.where(mp, 1, 0))
            plsc.store_scatter(pbuf, [offp + csp - 1], sp, mask=mp)
            return (offn + plsc.all_reduce_population_count(mn),
                    offp + plsc.all_reduce_population_count(mp))

        offn, offp = lax.fori_loop(0, nvec, p2, (jnp.zeros((_L16,), jnp.int32),
                                                 jnp.zeros((_L16,), jnp.int32)))
        cn = jnp.max(offn)
        cp = jnp.max(offp)
        plsc.store_scatter(nbuf, [cn + iota], _vec16(-_BIG))
        plsc.store_scatter(pbuf, [cp + iota], _vec16(_BIG))

        def casc_n(i, ts):
            s = jnp.sort(-nbuf[pl.ds(i * _L16, _L16)])
            for k in range(4):
                r = lax.rev(s, (0,))
                lo = jnp.minimum(ts[k], r)
                hi = jnp.maximum(ts[k], r)
                ts = ts[:k] + (jnp.sort(lo),) + ts[k + 1:]
                s = jnp.sort(hi)
            return ts

        tn = lax.fori_loop(0, (cn + _L16 - 1) >> 4, casc_n,
                           (_vec16(_BIG),) * 4)

        def casc_p(i, t0):
            s = jnp.sort(pbuf[pl.ds(i * _L16, _L16)])
            return jnp.sort(jnp.minimum(t0, lax.rev(s, (0,))))

        tp = lax.fori_loop(0, (cp + _L16 - 1) >> 4, casc_p, _vec16(_BIG))

        maxneg = -jnp.min(tn[0])
        for j in range(_NP):
            sp = _extract(tp, j, _BIG)
            mp_ = jnp.maximum(maxneg + 1.0 - sp, 0.0)
            num_v = _vec16(0.0)
            den_v = _vec16(0.0)
            for k in range(4):
                lossv = jnp.maximum(-tn[k] + (1.0 - sp), 0.0)
                prob = jnp.where(lossv > 0.0, lossv, _MNL)
                e = jnp.exp(prob - mp_)
                num_v = num_v + e * lossv
                den_v = den_v + e
            num = jnp.sum(num_v)
            den = jnp.sum(den_v)
            q = jnp.where(iota == 0, num, 0.0) / jnp.where(iota == 0, den, 1.0)
            acc = acc + jnp.where((iota == 0) & (num > 0.0), q, 0.0)
        return acc

    acc = lax.fori_loop(0, rpw, row_step, _vec16(0.0))
    obuf[...] = acc
    pltpu.sync_copy(obuf, out_hbm.at[pl.ds(wid * _L16, _L16)])


@jax.jit
def kernel(output, target):
    b, l = output.shape
    mesh = plsc.VectorSubcoreMesh(core_axis_name="c", subcore_axis_name="s")
    fn = functools.partial(
        pl.kernel,
        mesh=mesh,
        out_type=jax.ShapeDtypeStruct((32 * _L16,), jnp.float32),
        compiler_params=pltpu.CompilerParams(needs_layout_passes=False),
        scratch_types=[
            pltpu.VMEM((l,), jnp.float32),
            pltpu.VMEM((l,), jnp.float32),
            pltpu.VMEM((l,), jnp.float32),
            pltpu.VMEM((l,), jnp.float32),
            pltpu.VMEM((l + _L16,), jnp.float32),
            pltpu.VMEM((l + _L16,), jnp.float32),
            pltpu.VMEM((_L16,), jnp.float32),
        ],
    )(_sc_body)
    partials = fn(output, target)
    return jnp.sum(partials) / (b * _NP * _NN)


# speedup vs baseline: 14.3233x; 1.2514x over previous
import functools

import jax
import jax.numpy as jnp
from jax import lax
from jax.experimental import pallas as pl
from jax.experimental.pallas import tpu as pltpu
from jax.experimental.pallas import tpu_sc as plsc

_MXL = 100.0
_MNL = -100.0
_NP = 8
_NN = 64
_BIG = 3.4e38
_L16 = 16


def _vec16(x):
    return jnp.full((_L16,), x, jnp.float32)


_IOTA = lambda: lax.broadcasted_iota(jnp.int32, (_L16,), 0)


def _extract(v, j, pad):
    return jnp.min(jnp.where(_IOTA() == j, v, pad))


def _sc_body(x_hbm, t_hbm, out_hbm, xbuf, tbuf, smbuf, spbuf, nbuf, pbuf, obuf):
    b = x_hbm.shape[0]
    l = x_hbm.shape[1]
    nvec = l // _L16
    ngrp = 4
    gvec = nvec // ngrp
    wid = lax.axis_index("s") * 2 + lax.axis_index("c")
    rpw = b // 32
    iota = _IOTA()

    def row_step(r, acc):
        row = wid * rpw + r
        pltpu.sync_copy(x_hbm.at[row], xbuf)
        pltpu.sync_copy(t_hbm.at[row], tbuf)

        gmaxs = []
        pmin = _vec16(_BIG)
        for g in range(ngrp):
            @plsc.parallel_loop(0, gvec, unroll=8,
                                carry=(_vec16(-_BIG), pmin))
            def p1(i, carry, _g=g):
                gmax, pmin = carry
                xv = xbuf[pl.ds((_g * gvec + i) * _L16, _L16)]
                tv = tbuf[pl.ds((_g * gvec + i) * _L16, _L16)]
                m1 = tv == 1.0
                sm = jnp.where(m1, _MNL, xv)
                sp = jnp.where(m1, xv, _MXL)
                smbuf[pl.ds((_g * gvec + i) * _L16, _L16)] = sm
                spbuf[pl.ds((_g * gvec + i) * _L16, _L16)] = sp
                return jnp.maximum(gmax, sm), jnp.minimum(pmin, sp)

            gmax, pmin = p1
            gmaxs.append(gmax)
        tau_n = jnp.minimum(jnp.minimum(jnp.min(gmaxs[0]), jnp.min(gmaxs[1])),
                            jnp.minimum(jnp.min(gmaxs[2]), jnp.min(gmaxs[3])))
        tau_p = jnp.max(pmin)

        @plsc.parallel_loop(0, nvec, unroll=8,
                            carry=(jnp.int32(0), jnp.int32(0)))
        def p2(i, carry):
            offn, offp = carry
            sm = smbuf[pl.ds(i * _L16, _L16)]
            sp = spbuf[pl.ds(i * _L16, _L16)]
            mn = sm >= tau_n
            mp = sp <= tau_p
            plsc.store_compressed(nbuf.at[pl.ds(offn, _L16)], sm, mask=mn)
            plsc.store_compressed(pbuf.at[pl.ds(offp, _L16)], sp, mask=mp)
            return (offn + plsc.all_reduce_population_count(mn)[0],
                    offp + plsc.all_reduce_population_count(mp)[0])

        cn, cp = p2
        nbuf[pl.ds(cn, _L16)] = _vec16(-_BIG)
        pbuf[pl.ds(cp, _L16)] = _vec16(_BIG)

        def casc_n(i, ts):
            s = jnp.sort(-nbuf[pl.ds(i * _L16, _L16)])
            for k in range(4):
                r = lax.rev(s, (0,))
                lo = jnp.minimum(ts[k], r)
                hi = jnp.maximum(ts[k], r)
                ts = ts[:k] + (jnp.sort(lo),) + ts[k + 1:]
                s = jnp.sort(hi)
            return ts

        tn = lax.fori_loop(0, (cn + _L16 - 1) >> 4, casc_n,
                           (_vec16(_BIG),) * 4)

        def casc_p(i, t0):
            s = jnp.sort(pbuf[pl.ds(i * _L16, _L16)])
            return jnp.sort(jnp.minimum(t0, lax.rev(s, (0,))))

        tp = lax.fori_loop(0, (cp + _L16 - 1) >> 4, casc_p, _vec16(_BIG))

        maxneg = -jnp.min(tn[0])
        for j in range(_NP):
            sp = _extract(tp, j, _BIG)
            mp_ = jnp.maximum(maxneg + 1.0 - sp, 0.0)
            num_v = _vec16(0.0)
            den_v = _vec16(0.0)
            for k in range(4):
                lossv = jnp.maximum(-tn[k] + (1.0 - sp), 0.0)
                prob = jnp.where(lossv > 0.0, lossv, _MNL)
                e = jnp.exp(prob - mp_)
                num_v = num_v + e * lossv
                den_v = den_v + e
            num = jnp.sum(num_v)
            den = jnp.sum(den_v)
            q = jnp.where(iota == 0, num, 0.0) / jnp.where(iota == 0, den, 1.0)
            acc = acc + jnp.where((iota == 0) & (num > 0.0), q, 0.0)
        return acc

    acc = lax.fori_loop(0, rpw, row_step, _vec16(0.0))
    obuf[...] = acc
    pltpu.sync_copy(obuf, out_hbm.at[pl.ds(wid * _L16, _L16)])


@jax.jit
def kernel(output, target):
    b, l = output.shape
    mesh = plsc.VectorSubcoreMesh(core_axis_name="c", subcore_axis_name="s")
    fn = functools.partial(
        pl.kernel,
        mesh=mesh,
        out_type=jax.ShapeDtypeStruct((32 * _L16,), jnp.float32),
        compiler_params=pltpu.CompilerParams(needs_layout_passes=False),
        scratch_types=[
            pltpu.VMEM((l,), jnp.float32),
            pltpu.VMEM((l,), jnp.float32),
            pltpu.VMEM((l,), jnp.float32),
            pltpu.VMEM((l,), jnp.float32),
            pltpu.VMEM((l + _L16,), jnp.float32),
            pltpu.VMEM((l + _L16,), jnp.float32),
            pltpu.VMEM((_L16,), jnp.float32),
        ],
    )(_sc_body)
    partials = fn(output, target)
    return jnp.sum(partials) / (b * _NP * _NN)


# speedup vs baseline: 19.1409x; 1.3363x over previous
import functools

import jax
import jax.numpy as jnp
from jax import lax
from jax.experimental import pallas as pl
from jax.experimental.pallas import tpu as pltpu
from jax.experimental.pallas import tpu_sc as plsc

_MXL = 100.0
_MNL = -100.0
_NP = 8
_NN = 64
_BIG = 3.4e38
_L16 = 16


def _vec16(x):
    return jnp.full((_L16,), x, jnp.float32)


_IOTA = lambda: lax.broadcasted_iota(jnp.int32, (_L16,), 0)


def _extract(v, j, pad):
    return jnp.min(jnp.where(_IOTA() == j, v, pad))


def _sc_body(x_hbm, t_hbm, out_hbm, xbuf, tbuf, xbuf2, tbuf2,
             smbuf, spbuf, nbuf, pbuf, obuf, semx, semt, semx2, semt2):
    b = x_hbm.shape[0]
    l = x_hbm.shape[1]
    nvec = l // _L16
    ngrp = 4
    gvec = nvec // ngrp
    wid = lax.axis_index("s") * 2 + lax.axis_index("c")
    rpw = b // 32
    iota = _IOTA()

    def row_compute(acc, xbuf, tbuf):
        gmaxs = []
        pmin = _vec16(_BIG)
        for g in range(ngrp):
            @plsc.parallel_loop(0, gvec, unroll=8,
                                carry=(_vec16(-_BIG), pmin))
            def p1(i, carry, _g=g):
                gmax, pmin = carry
                xv = xbuf[pl.ds((_g * gvec + i) * _L16, _L16)]
                tv = tbuf[pl.ds((_g * gvec + i) * _L16, _L16)]
                m1 = tv == 1.0
                sm = jnp.where(m1, _MNL, xv)
                sp = jnp.where(m1, xv, _MXL)
                smbuf[pl.ds((_g * gvec + i) * _L16, _L16)] = sm
                spbuf[pl.ds((_g * gvec + i) * _L16, _L16)] = sp
                return jnp.maximum(gmax, sm), jnp.minimum(pmin, sp)

            gmax, pmin = p1
            gmaxs.append(gmax)
        tau_n = jnp.minimum(jnp.minimum(jnp.min(gmaxs[0]), jnp.min(gmaxs[1])),
                            jnp.minimum(jnp.min(gmaxs[2]), jnp.min(gmaxs[3])))
        tau_p = jnp.max(pmin)

        @plsc.parallel_loop(0, nvec, unroll=8,
                            carry=(jnp.int32(0), jnp.int32(0)))
        def p2(i, carry):
            offn, offp = carry
            sm = smbuf[pl.ds(i * _L16, _L16)]
            sp = spbuf[pl.ds(i * _L16, _L16)]
            mn = sm >= tau_n
            mp = sp <= tau_p
            plsc.store_compressed(nbuf.at[pl.ds(offn, _L16)], sm, mask=mn)
            plsc.store_compressed(pbuf.at[pl.ds(offp, _L16)], sp, mask=mp)
            return (offn + plsc.all_reduce_population_count(mn)[0],
                    offp + plsc.all_reduce_population_count(mp)[0])

        cn, cp = p2
        nbuf[pl.ds(cn, _L16)] = _vec16(-_BIG)
        pbuf[pl.ds(cp, _L16)] = _vec16(_BIG)

        def casc_n(i, ts):
            s = jnp.sort(-nbuf[pl.ds(i * _L16, _L16)])
            for k in range(4):
                r = lax.rev(s, (0,))
                lo = jnp.minimum(ts[k], r)
                hi = jnp.maximum(ts[k], r)
                ts = ts[:k] + (jnp.sort(lo),) + ts[k + 1:]
                s = jnp.sort(hi)
            return ts

        tn = lax.fori_loop(0, (cn + _L16 - 1) >> 4, casc_n,
                           (_vec16(_BIG),) * 4)

        def casc_p(i, t0):
            s = jnp.sort(pbuf[pl.ds(i * _L16, _L16)])
            return jnp.sort(jnp.minimum(t0, lax.rev(s, (0,))))

        tp = lax.fori_loop(0, (cp + _L16 - 1) >> 4, casc_p, _vec16(_BIG))

        maxneg = -jnp.min(tn[0])
        for j in range(_NP):
            sp = _extract(tp, j, _BIG)
            mp_ = jnp.maximum(maxneg + 1.0 - sp, 0.0)
            num_v = _vec16(0.0)
            den_v = _vec16(0.0)
            for k in range(4):
                lossv = jnp.maximum(-tn[k] + (1.0 - sp), 0.0)
                prob = jnp.where(lossv > 0.0, lossv, _MNL)
                e = jnp.exp(prob - mp_)
                num_v = num_v + e * lossv
                den_v = den_v + e
            num = jnp.sum(num_v)
            den = jnp.sum(den_v)
            q = jnp.where(iota == 0, num, 0.0) / jnp.where(iota == 0, den, 1.0)
            acc = acc + jnp.where((iota == 0) & (num > 0.0), q, 0.0)
        return acc

    def start(row, xb, tb, sx, st):
        pltpu.make_async_copy(x_hbm.at[row], xb, sx).start()
        pltpu.make_async_copy(t_hbm.at[row], tb, st).start()

    def wait(row, xb, tb, sx, st):
        pltpu.make_async_copy(x_hbm.at[row], xb, sx).wait()
        pltpu.make_async_copy(t_hbm.at[row], tb, st).wait()

    base = wid * rpw
    start(base, xbuf, tbuf, semx, semt)

    def pair_step(k, acc):
        ra = base + 2 * k
        rb = ra + 1
        start(rb, xbuf2, tbuf2, semx2, semt2)
        wait(ra, xbuf, tbuf, semx, semt)
        acc = row_compute(acc, xbuf, tbuf)
        start(jnp.minimum(ra + 2, base + rpw - 1), xbuf, tbuf, semx, semt)
        wait(rb, xbuf2, tbuf2, semx2, semt2)
        return row_compute(acc, xbuf2, tbuf2)

    acc = lax.fori_loop(0, rpw // 2, pair_step, _vec16(0.0))
    wait(base, xbuf, tbuf, semx, semt)
    obuf[...] = acc
    pltpu.sync_copy(obuf, out_hbm.at[pl.ds(wid * _L16, _L16)])


@jax.jit
def kernel(output, target):
    b, l = output.shape
    mesh = plsc.VectorSubcoreMesh(core_axis_name="c", subcore_axis_name="s")
    fn = functools.partial(
        pl.kernel,
        mesh=mesh,
        out_type=jax.ShapeDtypeStruct((32 * _L16,), jnp.float32),
        compiler_params=pltpu.CompilerParams(needs_layout_passes=False),
        scratch_types=[
            pltpu.VMEM((l,), jnp.float32),
            pltpu.VMEM((l,), jnp.float32),
            pltpu.VMEM((l,), jnp.float32),
            pltpu.VMEM((l,), jnp.float32),
            pltpu.VMEM((l,), jnp.float32),
            pltpu.VMEM((l,), jnp.float32),
            pltpu.VMEM((l + _L16,), jnp.float32),
            pltpu.VMEM((l + _L16,), jnp.float32),
            pltpu.VMEM((_L16,), jnp.float32),
            pltpu.SemaphoreType.DMA,
            pltpu.SemaphoreType.DMA,
            pltpu.SemaphoreType.DMA,
            pltpu.SemaphoreType.DMA,
        ],
    )(_sc_body)
    partials = fn(output, target)
    return jnp.sum(partials) / (b * _NP * _NN)
